# baseline (speedup 1.0000x reference)
import functools

import jax
import jax.numpy as jnp
from jax import lax
from jax.experimental import pallas as pl
from jax.experimental.pallas import tpu as pltpu
from jax.experimental.pallas import tpu_sc as plsc

B = 425984
D = 64
NC = 2
NS = 16
NW = NC * NS
BPW = B // NW
CH = 128
NCHUNK = BPW // CH
NPAIR = NCHUNK // 2

V = 1000000
NT = 7812
TPW = 244
GPW = TPW // 2
SR = 500032

_mesh = plsc.VectorSubcoreMesh(core_axis_name="c", subcore_axis_name="s")
_params = pltpu.CompilerParams(needs_layout_passes=False,
                               disable_bounds_checks=True)


def _transpose_tiles(tbuf, rbuf, ntile, lane):
    for t in range(ntile):
        for g in range(8):
            gl = lane + 16 * g
            rows = (gl >> 1) + 64 * t
            colsb = (gl & 1) << 6
            nii = 128 * t + 16 * g
            for a in range(8):
                vs = [tbuf[a, fi, pl.ds(nii, 16)] for fi in range(8)]
                for fi in range(8):
                    plsc.store_scatter(rbuf, [rows, colsb + (8 * a + fi)],
                                       vs[fi])


@functools.partial(
    pl.kernel,
    out_type=[jax.ShapeDtypeStruct((SR, 128), jnp.float32),
              jax.ShapeDtypeStruct((128, 128), jnp.float32)],
    mesh=_mesh,
    scratch_types=[
        [pltpu.VMEM((8, 8, 256), jnp.float32)] * 2,
        [pltpu.VMEM((128, 133), jnp.float32)] * 2,
        [pltpu.SemaphoreType.DMA] * 2,
        [pltpu.SemaphoreType.DMA] * 2,
    ],
    compiler_params=_params,
)
def _convert_kernel(table_hbm, tail_hbm, scr_hbm, dummy_hbm,
                    tbufs, rbufs, lsems, wsems):
    wid = lax.axis_index("s") * NC + lax.axis_index("c")
    g0 = wid * GPW
    lane = lax.iota(jnp.int32, 16)

    def fire_loads(g, c):
        tn0 = g * 2
        for a in range(8):
            pltpu.async_copy(
                table_hbm.at[pl.ds(8 * a, 8), pl.ds(128 * tn0, 256)],
                tbufs[c].at[a], lsems[c])

    def drain_loads(g, c):
        tn0 = g * 2
        for a in range(8):
            pltpu.make_async_copy(
                table_hbm.at[pl.ds(8 * a, 8), pl.ds(128 * tn0, 256)],
                tbufs[c].at[a], lsems[c]).wait()

    def fire_write(g, c):
        pltpu.async_copy(rbufs[c].at[:, pl.ds(0, 128)],
                         scr_hbm.at[pl.ds(128 * g, 128)], wsems[c])

    def drain_write(g, c):
        pltpu.make_async_copy(
            rbufs[c].at[:, pl.ds(0, 128)],
            scr_hbm.at[pl.ds(128 * g, 128)], wsems[c]).wait()

    fire_loads(g0, 0)
    fire_loads(g0 + 1, 1)
    pltpu.async_copy(rbufs[0].at[:, pl.ds(0, 128)], dummy_hbm, wsems[0])
    pltpu.async_copy(rbufs[1].at[:, pl.ds(0, 128)], dummy_hbm, wsems[1])

    def body(pp, carry):
        for c in range(2):
            g = g0 + 2 * pp + c
            drain_loads(g, c)
            drain_write(g, c)
            _transpose_tiles(tbufs[c], rbufs[c], 2, lane)
            fire_write(g, c)
            gn = jnp.minimum(g + 2, g0 + GPW - 1)
            fire_loads(gn, c)
        return carry

    lax.fori_loop(0, GPW // 2, body, 0)

    drain_loads(g0, 0)
    drain_loads(g0, 1)
    drain_write(g0, 0)
    drain_write(g0, 1)

    @pl.when(wid < 4)
    def _():
        tl = NT - 4 + wid
        for a in range(8):
            pltpu.async_copy(
                table_hbm.at[pl.ds(8 * a, 8), pl.ds(128 * tl, 128)],
                tbufs[0].at[a, :, pl.ds(0, 128)], lsems[0])
        for a in range(8):
            pltpu.make_async_copy(
                table_hbm.at[pl.ds(8 * a, 8), pl.ds(128 * tl, 128)],
                tbufs[0].at[a, :, pl.ds(0, 128)], lsems[0]).wait()
        _transpose_tiles(tbufs[0], rbufs[0], 1, lane)
        pltpu.async_copy(rbufs[0].at[pl.ds(0, 64), pl.ds(0, 128)],
                         scr_hbm.at[pl.ds(64 * tl, 64)], wsems[0])
        pltpu.make_async_copy(rbufs[0].at[pl.ds(0, 64), pl.ds(0, 128)],
                              scr_hbm.at[pl.ds(64 * tl, 64)], wsems[0]).wait()

    @pl.when(wid == 4)
    def _():
        pltpu.sync_copy(tail_hbm, rbufs[0].at[pl.ds(0, 32), pl.ds(0, 128)])
        pltpu.sync_copy(rbufs[0].at[pl.ds(0, 32), pl.ds(0, 128)],
                        scr_hbm.at[pl.ds(64 * NT, 32)])


@functools.partial(
    pl.kernel,
    out_type=[jax.ShapeDtypeStruct((D, B), jnp.float32),
              jax.ShapeDtypeStruct((D, CH), jnp.float32)],
    mesh=_mesh,
    scratch_types=[
        pltpu.VMEM((NCHUNK, CH), jnp.int32),
        pltpu.VMEM((NCHUNK, CH), jnp.int32),
        [pltpu.VMEM((CH, 128), jnp.float32)] * 2,
        [pltpu.VMEM((D, CH), jnp.float32)] * 2,
        [pltpu.SemaphoreType.DMA] * 2,
        [pltpu.SemaphoreType.DMA] * 2,
    ],
    compiler_params=_params,
)
def _gather_kernel(rows_hbm, base_hbm, scr_hbm, out_hbm, dummy_hbm,
                   rows_v, base_v, gbufs, stages, gsems, wsems):
    wid = lax.axis_index("s") * NC + lax.axis_index("c")
    pos0 = wid * BPW
    pltpu.sync_copy(rows_hbm.at[wid], rows_v)
    pltpu.sync_copy(base_hbm.at[wid], base_v)

    lane = lax.iota(jnp.int32, 16)

    def fire_gather(j, c):
        pltpu.async_copy(scr_hbm.at[rows_v.at[j]], gbufs[c], gsems[c])

    def drain_gather(j, c):
        pltpu.make_async_copy(
            scr_hbm.at[rows_v.at[j]], gbufs[c], gsems[c]).wait()

    def fire_write(j, c):
        pltpu.async_copy(
            stages[c], out_hbm.at[:, pl.ds(pos0 + j * CH, CH)], wsems[c])

    def drain_write(j, c):
        pltpu.make_async_copy(
            stages[c], out_hbm.at[:, pl.ds(pos0 + j * CH, CH)],
            wsems[c]).wait()

    fire_gather(0, 0)
    fire_gather(1, 1)
    pltpu.async_copy(stages[0], dummy_hbm, wsems[0])
    pltpu.async_copy(stages[1], dummy_hbm, wsems[1])

    def body(jj, carry):
        for c in range(2):
            j = 2 * jj + c
            drain_gather(j, c)
            drain_write(j, c)
            for g in range(8):
                ivec = lane + 16 * g
                cv = base_v[j, pl.ds(16 * g, 16)]
                for fb in range(4):
                    vs = [plsc.load_gather(gbufs[c], [ivec, cv + (16 * fb + df)])
                          for df in range(16)]
                    for df in range(16):
                        stages[c][16 * fb + df, pl.ds(16 * g, 16)] = vs[df]
            fire_write(j, c)
            fire_gather(jnp.minimum(j + 2, NCHUNK - 1), c)
        return carry

    lax.fori_loop(0, NPAIR, body, 0)

    drain_gather(0, 0)
    drain_gather(0, 1)
    drain_write(0, 0)
    drain_write(0, 1)


def kernel(nodes, kg_features):
    n = nodes.astype(jnp.int32)
    rows = (n >> 1).reshape(NW, NCHUNK, CH)
    base = ((n & 1) << 6).reshape(NW, NCHUNK, CH)
    table_t = kg_features.T
    tail = kg_features[128 * NT:].reshape(32, 128)
    scratch, _ = _convert_kernel(table_t, tail)
    out_t, _ = _gather_kernel(rows, base, scratch)
    return out_t.T


# speedup vs baseline: 1.4491x; 1.4491x over previous
import functools

import jax
import jax.numpy as jnp
from jax import lax
from jax.experimental import pallas as pl
from jax.experimental.pallas import tpu as pltpu
from jax.experimental.pallas import tpu_sc as plsc

B = 425984
D = 64
NC = 2
NS = 16
NW = NC * NS
BPW = B // NW
CH = 128
NCHUNK = BPW // CH
K = 4
NG = NCHUNK // K
NP = NG // 2

_mesh = plsc.VectorSubcoreMesh(core_axis_name="c", subcore_axis_name="s")


@functools.partial(
    pl.kernel,
    out_type=jax.ShapeDtypeStruct((B, D), jnp.float32),
    mesh=_mesh,
    scratch_types=[
        pltpu.VMEM((NCHUNK, CH), jnp.int32),
        pltpu.VMEM((K, CH, D), jnp.float32),
        pltpu.VMEM((K, CH, D), jnp.float32),
        pltpu.SemaphoreType.DMA,
        pltpu.SemaphoreType.DMA,
        pltpu.SemaphoreType.DMA,
        pltpu.SemaphoreType.DMA,
    ],
    compiler_params=pltpu.CompilerParams(use_tc_tiling_on_sc=False),
)
def _gather_kernel(idx_hbm, table_hbm, out_hbm, idx_v, rows0, rows1,
                   gsem0, gsem1, wsem0, wsem1):
    wid = lax.axis_index("s") * NC + lax.axis_index("c")
    base = wid * BPW
    pltpu.sync_copy(idx_hbm.at[wid], idx_v)

    def fire_gathers(g, rows, gsem):
        for b in range(K):
            pltpu.async_copy(table_hbm.at[idx_v.at[g * K + b]], rows.at[b], gsem)

    def drain_gathers(g, rows, gsem):
        for b in range(K):
            pltpu.make_async_copy(
                table_hbm.at[idx_v.at[g * K + b]], rows.at[b], gsem).wait()

    def fire_writes(g, rows, wsem):
        for b in range(K):
            pltpu.async_copy(
                rows.at[b], out_hbm.at[pl.ds(base + (g * K + b) * CH, CH)], wsem)

    def drain_writes(g, rows, wsem):
        for b in range(K):
            pltpu.make_async_copy(
                rows.at[b], out_hbm.at[pl.ds(base + (g * K + b) * CH, CH)],
                wsem).wait()

    fire_gathers(0, rows0, gsem0)
    fire_gathers(1, rows1, gsem1)

    def body(p, carry):
        drain_gathers(2 * p, rows0, gsem0)
        fire_writes(2 * p, rows0, wsem0)
        drain_gathers(2 * p + 1, rows1, gsem1)
        fire_writes(2 * p + 1, rows1, wsem1)
        drain_writes(2 * p, rows0, wsem0)
        fire_gathers(2 * p + 2, rows0, gsem0)
        drain_writes(2 * p + 1, rows1, wsem1)
        fire_gathers(2 * p + 3, rows1, gsem1)
        return carry

    lax.fori_loop(0, NP - 1, body, 0)

    p = NP - 1
    drain_gathers(2 * p, rows0, gsem0)
    fire_writes(2 * p, rows0, wsem0)
    drain_gathers(2 * p + 1, rows1, gsem1)
    fire_writes(2 * p + 1, rows1, wsem1)
    drain_writes(2 * p, rows0, wsem0)
    drain_writes(2 * p + 1, rows1, wsem1)


def kernel(nodes, kg_features):
    idx = nodes.astype(jnp.int32).reshape(NW, NCHUNK, CH)
    return _gather_kernel(idx, kg_features)
